# baseline (speedup 1.0000x reference)
import functools

import jax
import jax.numpy as jnp
from jax import lax
from jax.experimental import pallas as pl
from jax.experimental.pallas import tpu as pltpu
from jax.experimental.pallas import tpu_sc as plsc

_EMB = 32
_B = 16384
_NF = 19
_NC = 2
_NS = 16
_NW = _NC * _NS
_BPW = _B // _NW


def _body(xt_hbm, *refs):
    tables = refs[:_NF]
    out_hbm = refs[_NF]
    idx_v, rows_v, sem = refs[_NF + 1:]
    wid = lax.axis_index("s") * _NC + lax.axis_index("c")
    base = wid * _BPW
    for i in range(_NF):
        pltpu.sync_copy(xt_hbm.at[pl.ds(i * _B + base, _BPW)], idx_v)
        pltpu.async_copy(tables[i].at[idx_v], rows_v, sem).wait()
        pltpu.sync_copy(rows_v,
                        out_hbm.at[pl.ds(base, _BPW), pl.ds(i * _EMB, _EMB)])


_sc_lookup = functools.partial(
    pl.kernel,
    out_type=jax.ShapeDtypeStruct((_B, _NF * _EMB), jnp.float32),
    mesh=plsc.VectorSubcoreMesh(core_axis_name="c", subcore_axis_name="s"),
    compiler_params=pltpu.CompilerParams(use_tc_tiling_on_sc=False),
    scratch_types=[
        pltpu.VMEM((_BPW,), jnp.int32),
        pltpu.VMEM((_BPW, _EMB), jnp.float32),
        pltpu.SemaphoreType.DMA,
    ],
)(_body)


def kernel(x, W0, W1, W2, W3, W4, W5, W6, W7, W8, W9, W10, W11, W12, W13,
           W14, W15, W16, W17, W18):
    xt = x.T.reshape(-1)
    out = _sc_lookup(xt, W0, W1, W2, W3, W4, W5, W6, W7, W8, W9, W10, W11,
                     W12, W13, W14, W15, W16, W17, W18)
    return out.reshape(_B, _NF, _EMB)


# speedup vs baseline: 1.0240x; 1.0240x over previous
import functools

import jax
import jax.numpy as jnp
from jax import lax
from jax.experimental import pallas as pl
from jax.experimental.pallas import tpu as pltpu
from jax.experimental.pallas import tpu_sc as plsc

_EMB = 32
_B = 16384
_NF = 19
_NC = 2
_NS = 16
_NW = _NC * _NS
_BPW = _B // _NW


_NBUF = 6


def _body(xt_hbm, *refs):
    tables = refs[:_NF]
    out_hbm = refs[_NF]
    idx_v, rows_v, gsem, wsem = refs[_NF + 1:]
    wid = lax.axis_index("s") * _NC + lax.axis_index("c")
    base = wid * _BPW
    pltpu.sync_copy(xt_hbm.at[:, pl.ds(base, _BPW)], idx_v)

    def gather(i):
        return pltpu.async_copy(tables[i].at[idx_v.at[i]],
                                rows_v.at[i % _NBUF], gsem)

    def write(i):
        return pltpu.async_copy(
            rows_v.at[i % _NBUF],
            out_hbm.at[pl.ds(base, _BPW), pl.ds(i * _EMB, _EMB)], wsem)

    gd = [gather(i) for i in range(_NBUF)]
    wd = []
    for i in range(_NF):
        gd[i].wait()
        wd.append(write(i))
        j = i + _NBUF
        if j < _NF:
            wd[i].wait()
            gd.append(gather(j))
    for i in range(_NF - _NBUF, _NF):
        wd[i].wait()


_sc_lookup = functools.partial(
    pl.kernel,
    out_type=jax.ShapeDtypeStruct((_B, _NF * _EMB), jnp.float32),
    mesh=plsc.VectorSubcoreMesh(core_axis_name="c", subcore_axis_name="s"),
    compiler_params=pltpu.CompilerParams(use_tc_tiling_on_sc=False),
    scratch_types=[
        pltpu.VMEM((_NF, _BPW), jnp.int32),
        pltpu.VMEM((_NBUF, _BPW, _EMB), jnp.float32),
        pltpu.SemaphoreType.DMA,
        pltpu.SemaphoreType.DMA,
    ],
)(_body)


def kernel(x, W0, W1, W2, W3, W4, W5, W6, W7, W8, W9, W10, W11, W12, W13,
           W14, W15, W16, W17, W18):
    xt = x.T
    out = _sc_lookup(xt, W0, W1, W2, W3, W4, W5, W6, W7, W8, W9, W10, W11,
                     W12, W13, W14, W15, W16, W17, W18)
    return out.reshape(_B, _NF, _EMB)
